# baseline (speedup 1.0000x reference)
import functools

import jax
import jax.numpy as jnp
from jax import lax
from jax.experimental import pallas as pl
from jax.experimental.pallas import tpu as pltpu
from jax.experimental.pallas import tpu_sc as plsc

S = 2048
B = 4
D = 1024
N_TOK = S * B
L = 16
NC = 2
NS = 16
NW = NC * NS
TOK_PER_W = N_TOK // NW
CHUNK = 64
N_CHUNKS = TOK_PER_W // CHUNK
POS_PER_CHUNK = CHUNK // B
DJ = D // L

_MESH = plsc.VectorSubcoreMesh(core_axis_name="c", subcore_axis_name="s")


@functools.partial(
    pl.kernel,
    mesh=_MESH,
    compiler_params=pltpu.CompilerParams(needs_layout_passes=False),
    out_type=jax.ShapeDtypeStruct((N_TOK, D), jnp.float32),
    scratch_types=[
        pltpu.VMEM((CHUNK,), jnp.int32),
        pltpu.VMEM((CHUNK, D), jnp.float32),
        pltpu.VMEM((POS_PER_CHUNK, D), jnp.float32),
        pltpu.VMEM((D,), jnp.float32),
        pltpu.VMEM((D,), jnp.float32),
        pltpu.SemaphoreType.DMA,
    ],
)
def _emb_ln(x_hbm, tok_hbm, pos_hbm, gamma_hbm, beta_hbm, out_hbm,
            idx_v, h_v, pos_v, gamma_v, beta_v, sem):
    wid = lax.axis_index("s") * NC + lax.axis_index("c")
    base = wid * TOK_PER_W

    pltpu.sync_copy(gamma_hbm, gamma_v)
    pltpu.sync_copy(beta_hbm, beta_v)

    for c in range(N_CHUNKS):
        cbase = pl.multiple_of(base + c * CHUNK, CHUNK)
        pltpu.sync_copy(x_hbm.at[pl.ds(cbase, CHUNK)], idx_v)
        pltpu.sync_copy(
            pos_hbm.at[pl.ds(pl.multiple_of(cbase // B, POS_PER_CHUNK),
                             POS_PER_CHUNK)], pos_v)
        pltpu.async_copy(tok_hbm.at[idx_v], h_v, sem).wait()

        def row_body(r, carry):
            pr = r // B
            s_acc = jnp.zeros((L,), jnp.float32)
            q_acc = jnp.zeros((L,), jnp.float32)
            for j in range(DJ):
                t = h_v[r, pl.ds(j * L, L)]
                p = pos_v[pr, pl.ds(j * L, L)]
                h = t + p
                h_v[r, pl.ds(j * L, L)] = h
                s_acc = s_acc + h
                q_acc = q_acc + h * h
            tot = jnp.sum(s_acc)
            tot2 = jnp.sum(q_acc)
            mean = tot * (1.0 / D)
            var = tot2 * (1.0 / D) - mean * mean
            xe = var + 1e-5
            iv = lax.bitcast_convert_type(xe, jnp.int32)
            y = lax.bitcast_convert_type(
                jnp.int32(0x5F3759DF) - lax.shift_right_logical(iv, 1),
                jnp.float32)
            for _ in range(3):
                y = y * (1.5 - 0.5 * xe * y * y)
            rstd = y
            for j in range(DJ):
                h = h_v[r, pl.ds(j * L, L)]
                g = gamma_v[pl.ds(j * L, L)]
                bta = beta_v[pl.ds(j * L, L)]
                h_v[r, pl.ds(j * L, L)] = (h - mean) * rstd * g + bta
            return carry

        lax.fori_loop(0, CHUNK, row_body, 0)
        pltpu.sync_copy(h_v, out_hbm.at[pl.ds(cbase, CHUNK)])


def kernel(x, tok_table, pos_table, gamma, beta):
    xf = x.reshape(-1).astype(jnp.int32)
    out = _emb_ln(xf, tok_table, pos_table, gamma, beta)
    return out.reshape(S, B, D)


# speedup vs baseline: 2.4476x; 2.4476x over previous
import functools

import jax
import jax.numpy as jnp
from jax import lax
from jax.experimental import pallas as pl
from jax.experimental.pallas import tpu as pltpu
from jax.experimental.pallas import tpu_sc as plsc

S = 2048
B = 4
D = 1024
N_TOK = S * B
NC = 2
NS = 16
NW = NC * NS
TOK_PER_W = N_TOK // NW
GCH = 32
NCH = TOK_PER_W // GCH

_MESH = plsc.VectorSubcoreMesh(core_axis_name="c", subcore_axis_name="s")


@functools.partial(
    pl.kernel,
    mesh=_MESH,
    compiler_params=pltpu.CompilerParams(needs_layout_passes=False),
    out_type=jax.ShapeDtypeStruct((N_TOK, D), jnp.float32),
    scratch_types=[
        pltpu.VMEM((NCH, GCH), jnp.int32),
        pltpu.VMEM((2, GCH, D), jnp.float32),
        pltpu.SemaphoreType.DMA,
        pltpu.SemaphoreType.DMA,
        pltpu.SemaphoreType.DMA,
        pltpu.SemaphoreType.DMA,
    ],
)
def _sc_gather(x_hbm, tok_hbm, out_hbm, idx_v, buf_v, g0, g1, o0, o1):
    wid = lax.axis_index("s") * NC + lax.axis_index("c")
    base = wid * TOK_PER_W
    gsem = (g0, g1)
    osem = (o0, o1)

    pltpu.sync_copy(x_hbm.at[wid], idx_v)

    gath = [None, None]
    outc = [None, None]
    for c in range(NCH):
        slot = c & 1
        if outc[slot] is not None:
            outc[slot].wait()
        gath[slot] = pltpu.async_copy(
            tok_hbm.at[idx_v.at[c]], buf_v.at[slot], gsem[slot])
        prev = slot ^ 1
        if gath[prev] is not None:
            gath[prev].wait()
            obase = pl.multiple_of(base + (c - 1) * GCH, GCH)
            outc[prev] = pltpu.async_copy(
                buf_v.at[prev], out_hbm.at[pl.ds(obase, GCH)], osem[prev])
    last = (NCH - 1) & 1
    gath[last].wait()
    obase = pl.multiple_of(base + (NCH - 1) * GCH, GCH)
    outc[last] = pltpu.async_copy(
        buf_v.at[last], out_hbm.at[pl.ds(obase, GCH)], osem[last])
    outc[last ^ 1].wait()
    outc[last].wait()


RS = 256


def _tc_ln_body(h_ref, pos_ref, g_ref, b_ref, o_ref):
    h = h_ref[...] + pos_ref[...]
    mean = jnp.mean(h, axis=-1, keepdims=True)
    cent = h - mean
    var = jnp.mean(cent * cent, axis=-1, keepdims=True)
    o_ref[...] = cent * lax.rsqrt(var + 1e-5) * g_ref[...] + b_ref[...]


_tc_ln = pl.pallas_call(
    _tc_ln_body,
    grid=(S // RS,),
    in_specs=[
        pl.BlockSpec((RS, B, D), lambda i: (i, 0, 0)),
        pl.BlockSpec((RS, 1, D), lambda i: (i, 0, 0)),
        pl.BlockSpec((1, 1, D), lambda i: (0, 0, 0)),
        pl.BlockSpec((1, 1, D), lambda i: (0, 0, 0)),
    ],
    out_specs=pl.BlockSpec((RS, B, D), lambda i: (i, 0, 0)),
    out_shape=jax.ShapeDtypeStruct((S, B, D), jnp.float32),
    compiler_params=pltpu.CompilerParams(
        dimension_semantics=("arbitrary",),
    ),
)


def kernel(x, tok_table, pos_table, gamma, beta):
    xf = x.reshape(NW, NCH, GCH).astype(jnp.int32)
    rows = _sc_gather(xf, tok_table)
    return _tc_ln(
        rows.reshape(S, B, D),
        pos_table.reshape(S, 1, D),
        gamma.reshape(1, 1, D),
        beta.reshape(1, 1, D),
    )
